# Initial kernel scaffold; baseline (speedup 1.0000x reference)
import jax
import jax.numpy as jnp
from jax.experimental import pallas as pl


def kernel(x, coeffs):
    raise NotImplementedError("write your pallas kernel here")


# speedup vs baseline: 103.0588x; 103.0588x over previous
import functools
import jax
import jax.numpy as jnp
from jax import lax
from jax.experimental import pallas as pl
from jax.experimental.pallas import tpu as pltpu
from jax.experimental.pallas import tpu_sc as plsc

NPTS = 4194304
RGRID = 513
NW = 32
PPW = NPTS // NW
CHUNK = 2048
NCH = PPW // CHUNK
NG = CHUNK // 16
SLEN = 128
NSTR = CHUNK // SLEN


def _spline_body(xf, p16, out, xv, f0b, f1b, idxb, patches, outb, sem):
    cid = lax.axis_index("c")
    sid = lax.axis_index("s")
    wid = sid * 2 + cid
    lane = lax.iota(jnp.int32, 16)

    def chunk_body(g, _):
        base = wid * PPW + g * CHUNK
        pltpu.sync_copy(xf.at[pl.ds(base * 2, CHUNK * 2)], xv)

        def idx_body(i, _):
            off = i * 32
            x0 = plsc.load_gather(xv, [off + 2 * lane])
            x1 = plsc.load_gather(xv, [off + 2 * lane + 1])
            t0 = x0 * 512.0 + 0.5
            t1 = x1 * 512.0 + 0.5
            r0 = t0.astype(jnp.int32)
            r1 = t1.astype(jnp.int32)
            f0 = t0 - r0.astype(jnp.float32)
            f1 = t1 - r1.astype(jnp.float32)
            f0b[pl.ds(i * 16, 16)] = f0
            f1b[pl.ds(i * 16, 16)] = f1
            idxb[pl.ds(i * 16, 16)] = r0 * RGRID + r1
            return 0

        lax.fori_loop(0, NG, idx_body, 0)

        copies = []
        for j in range(NSTR):
            cp = pltpu.async_copy(
                p16.at[idxb.at[pl.ds(j * SLEN, SLEN)]],
                patches.at[pl.ds(j * SLEN, SLEN)],
                sem,
            )
            copies.append(cp)
        for cp in copies:
            cp.wait()

        def out_body(i, _):
            f0 = f0b[pl.ds(i * 16, 16)]
            f1 = f1b[pl.ds(i * 16, 16)]
            u0 = 1.0 - f0
            u1 = 1.0 - f1
            f0sq = f0 * f0
            f1sq = f1 * f1
            u0sq = u0 * u0
            u1sq = u1 * u1
            b1 = (u0sq * u0, (3.0 * f0 - 6.0) * f0sq + 4.0,
                  (3.0 * u0 - 6.0) * u0sq + 4.0, f0sq * f0)
            b2 = (u1sq * u1, (3.0 * f1 - 6.0) * f1sq + 4.0,
                  (3.0 * u1 - 6.0) * u1sq + 4.0, f1sq * f1)
            row = i * 16 + lane
            acc = jnp.zeros((16,), jnp.float32)
            for j1 in range(4):
                col = jnp.full((16,), j1 * 4, jnp.int32)
                racc = plsc.load_gather(patches, [row, col]) * b2[0]
                for j2 in range(1, 4):
                    col = jnp.full((16,), j1 * 4 + j2, jnp.int32)
                    racc = racc + plsc.load_gather(patches, [row, col]) * b2[j2]
                acc = acc + racc * b1[j1]
            outb[pl.ds(i * 16, 16)] = acc
            return 0

        lax.fori_loop(0, NG, out_body, 0)
        pltpu.sync_copy(outb, out.at[pl.ds(base, CHUNK)])
        return 0

    lax.fori_loop(0, NCH, chunk_body, 0)


@jax.jit
def _run(xf, p16):
    mesh = plsc.VectorSubcoreMesh(core_axis_name="c", subcore_axis_name="s")
    f = pl.kernel(
        _spline_body,
        out_type=jax.ShapeDtypeStruct((NPTS,), jnp.float32),
        mesh=mesh,
        scratch_types=[
            pltpu.VMEM((CHUNK * 2,), jnp.float32),
            pltpu.VMEM((CHUNK,), jnp.float32),
            pltpu.VMEM((CHUNK,), jnp.float32),
            pltpu.VMEM((CHUNK,), jnp.int32),
            pltpu.VMEM((CHUNK, 16), jnp.float32),
            pltpu.VMEM((CHUNK,), jnp.float32),
            pltpu.SemaphoreType.DMA,
        ],
        compiler_params=pltpu.CompilerParams(
            needs_layout_passes=False, use_tc_tiling_on_sc=False),
    )
    return f(xf, p16)


def kernel(x, coeffs):
    p16 = jnp.stack(
        [coeffs[i:i + RGRID, j:j + RGRID] for i in range(4) for j in range(4)],
        axis=-1,
    ).reshape(RGRID * RGRID, 16)
    out = _run(x.reshape(-1), p16)
    return out.reshape(NPTS, 1)


# speedup vs baseline: 104.5063x; 1.0140x over previous
import functools
import jax
import jax.numpy as jnp
from jax import lax
from jax.experimental import pallas as pl
from jax.experimental.pallas import tpu as pltpu
from jax.experimental.pallas import tpu_sc as plsc

NPTS = 4194304
RGRID = 513
NPATCH = RGRID * RGRID
CROWS = 2088
NW = 32
PPW = NPTS // NW
CHUNK = 2048
NCH = PPW // CHUNK
NG = CHUNK // 16
SLEN = 128
NSTR = CHUNK // SLEN
RPT = 33


def _spline_body(xf, cf2, out, p16, xv, f0b, f1b, idxb, patches, outb,
                 cbuf, pbuf, sem):
    cid = lax.axis_index("c")
    sid = lax.axis_index("s")
    wid = sid * 2 + cid
    roff = cid * NPATCH
    lane = lax.iota(jnp.int32, 16)
    patc = (lane >> 2) * 516 + (lane & 3)

    def build_r(t, _):
        r = sid * RPT + t

        @pl.when(r < RGRID)
        def _():
            fstart = r * 516
            sr = fstart >> 7
            rel0 = fstart - sr * 128
            pltpu.sync_copy(cf2.at[pl.ds(sr, 18), :], cbuf)

            def build_c(c, _):
                rel = rel0 + c + patc
                v = plsc.load_gather(cbuf, [rel >> 7, rel & 127])
                pbuf[c, :] = v
                return 0

            lax.fori_loop(0, RGRID, build_c, 0)
            pltpu.sync_copy(pbuf, p16.at[pl.ds(roff + r * RGRID, RGRID), :])

        return 0

    lax.fori_loop(0, RPT, build_r, 0)
    plsc.subcore_barrier()

    def chunk_body(g, _):
        base = wid * PPW + g * CHUNK
        pltpu.sync_copy(xf.at[pl.ds(base * 2, CHUNK * 2)], xv)

        def idx_body(i, _):
            off = i * 32
            x0 = plsc.load_gather(xv, [off + 2 * lane])
            x1 = plsc.load_gather(xv, [off + 2 * lane + 1])
            t0 = x0 * 512.0 + 0.5
            t1 = x1 * 512.0 + 0.5
            r0 = t0.astype(jnp.int32)
            r1 = t1.astype(jnp.int32)
            f0 = t0 - r0.astype(jnp.float32)
            f1 = t1 - r1.astype(jnp.float32)
            f0b[pl.ds(i * 16, 16)] = f0
            f1b[pl.ds(i * 16, 16)] = f1
            idxb[pl.ds(i * 16, 16)] = r0 * RGRID + r1 + roff
            return 0

        lax.fori_loop(0, NG, idx_body, 0)

        copies = []
        for j in range(NSTR):
            cp = pltpu.async_copy(
                p16.at[idxb.at[pl.ds(j * SLEN, SLEN)]],
                patches.at[pl.ds(j * SLEN, SLEN)],
                sem,
            )
            copies.append(cp)
        for cp in copies:
            cp.wait()

        def out_body(i, _):
            f0 = f0b[pl.ds(i * 16, 16)]
            f1 = f1b[pl.ds(i * 16, 16)]
            u0 = 1.0 - f0
            u1 = 1.0 - f1
            f0sq = f0 * f0
            f1sq = f1 * f1
            u0sq = u0 * u0
            u1sq = u1 * u1
            b1 = (u0sq * u0, (3.0 * f0 - 6.0) * f0sq + 4.0,
                  (3.0 * u0 - 6.0) * u0sq + 4.0, f0sq * f0)
            b2 = (u1sq * u1, (3.0 * f1 - 6.0) * f1sq + 4.0,
                  (3.0 * u1 - 6.0) * u1sq + 4.0, f1sq * f1)
            row = i * 16 + lane
            acc = jnp.zeros((16,), jnp.float32)
            for j1 in range(4):
                col = jnp.full((16,), j1 * 4, jnp.int32)
                racc = plsc.load_gather(patches, [row, col]) * b2[0]
                for j2 in range(1, 4):
                    col = jnp.full((16,), j1 * 4 + j2, jnp.int32)
                    racc = racc + plsc.load_gather(patches, [row, col]) * b2[j2]
                acc = acc + racc * b1[j1]
            outb[pl.ds(i * 16, 16)] = acc
            return 0

        lax.fori_loop(0, NG, out_body, 0)
        pltpu.sync_copy(outb, out.at[pl.ds(base, CHUNK)])
        return 0

    lax.fori_loop(0, NCH, chunk_body, 0)


@jax.jit
def _run(xf, cf2):
    mesh = plsc.VectorSubcoreMesh(core_axis_name="c", subcore_axis_name="s")
    f = pl.kernel(
        _spline_body,
        out_type=jax.ShapeDtypeStruct((NPTS,), jnp.float32),
        mesh=mesh,
        scratch_types=[
            pltpu.HBM((2 * NPATCH, 16), jnp.float32),
            pltpu.VMEM((CHUNK * 2,), jnp.float32),
            pltpu.VMEM((CHUNK,), jnp.float32),
            pltpu.VMEM((CHUNK,), jnp.float32),
            pltpu.VMEM((CHUNK,), jnp.int32),
            pltpu.VMEM((CHUNK, 16), jnp.float32),
            pltpu.VMEM((CHUNK,), jnp.float32),
            pltpu.VMEM((18, 128), jnp.float32),
            pltpu.VMEM((RGRID, 16), jnp.float32),
            pltpu.SemaphoreType.DMA,
        ],
        compiler_params=pltpu.CompilerParams(
            needs_layout_passes=False, use_tc_tiling_on_sc=False),
    )
    return f(xf, cf2)


def kernel(x, coeffs):
    cf2 = jnp.pad(coeffs.reshape(-1), (0, CROWS * 128 - 516 * 516))
    cf2 = cf2.reshape(CROWS, 128)
    out = _run(x.reshape(-1), cf2)
    return out.reshape(NPTS, 1)
